# Initial kernel scaffold; baseline (speedup 1.0000x reference)
import jax
import jax.numpy as jnp
from jax.experimental import pallas as pl


def kernel(x, edge_index, pan_w1, W1, b1, pan_w2, W2, b2):
    raise NotImplementedError("write your pallas kernel here")


# speedup vs baseline: 4.6568x; 4.6568x over previous
import jax
import jax.numpy as jnp
from jax import lax
from jax.experimental import pallas as pl
from jax.experimental.pallas import tpu as pltpu
from jax.experimental.pallas import tpu_sc as plsc

_N = 10000
_NP = 10240
_E = 320000
_NS = 16
_NC = 2
_CW = 128
_CH = 160
_EP = _NS * _CH * _CW
_RPT = _NP // _NS
_D = 128
_DH = 64
_F32 = jnp.float32


def _mesh():
    return plsc.VectorSubcoreMesh(
        core_axis_name="c", subcore_axis_name="s",
        num_cores=_NC, num_subcores=_NS)


def _make_spmm(width, split):
    def body(*refs):
        if split:
            ylo, yhi, rowp, colp, olo, ohi, idxr, idxc, gath, obuf, acc, sem = refs
        else:
            y, rowp, colp, out, idxr, idxc, gath, obuf, acc, sem = refs
        c = lax.axis_index("c")
        s = lax.axis_index("s")
        zeros16 = jnp.zeros((16,), _F32)

        pltpu.sync_copy(rowp.at[s], idxr)
        pltpu.sync_copy(colp.at[s], idxc)

        def zrow(i, carry):
            for k in range(width // 16):
                obuf[i, pl.ds(k * 16, 16)] = zeros16
            return carry
        lax.fori_loop(0, _RPT, zrow, None)
        pltpu.sync_copy(obuf, acc.at[pl.ds(s * _RPT, _RPT)])
        plsc.subcore_barrier()

        def run(src):
            def b(j, carry):
                pltpu.async_copy(src.at[idxc.at[j]], gath, sem).wait()
                pltpu.sync_copy(gath, acc.at[idxr.at[j]], add=True)
                return carry
            lax.fori_loop(0, _CH, b, None)

        if split:
            @pl.when(c == 0)
            def _():
                run(ylo)

            @pl.when(c == 1)
            def _():
                run(yhi)
        else:
            run(y)

        plsc.subcore_barrier()
        pltpu.sync_copy(acc.at[pl.ds(s * _RPT, _RPT)], obuf)

        if split:
            @pl.when(c == 0)
            def _():
                pltpu.sync_copy(obuf, olo.at[pl.ds(s * _RPT, _RPT)])

            @pl.when(c == 1)
            def _():
                pltpu.sync_copy(obuf, ohi.at[pl.ds(s * _RPT, _RPT)])
        else:
            @pl.when(c == 0)
            def _():
                pltpu.sync_copy(obuf, out.at[pl.ds(s * _RPT, _RPT)])

    n_out = 2 if split else 1
    out_type = tuple(jax.ShapeDtypeStruct((_NP, width), _F32)
                     for _ in range(n_out))
    if not split:
        out_type = out_type[0]
    return pl.kernel(
        body,
        out_type=out_type,
        mesh=_mesh(),
        compiler_params=pltpu.CompilerParams(use_tc_tiling_on_sc=False),
        scratch_types=[
            pltpu.VMEM((_CH, _CW), jnp.int32),
            pltpu.VMEM((_CH, _CW), jnp.int32),
            pltpu.VMEM((_CW, width), _F32),
            pltpu.VMEM((_RPT, width), _F32),
            pltpu.VMEM_SHARED((_NP, width), _F32),
            pltpu.SemaphoreType.DMA,
        ],
    )


_BR = 1024


def _lin_relu(a, W, b):
    def body(a_ref, w_ref, b_ref, o_ref):
        t = jnp.dot(a_ref[...], w_ref[...], preferred_element_type=_F32)
        o_ref[...] = jnp.maximum(t + b_ref[...], 0.0)

    return pl.pallas_call(
        body,
        grid=(_NP // _BR,),
        in_specs=[pl.BlockSpec((_BR, _D), lambda i: (i, 0)),
                  pl.BlockSpec((_D, _D), lambda i: (0, 0)),
                  pl.BlockSpec((1, _D), lambda i: (0, 0))],
        out_specs=pl.BlockSpec((_BR, _D), lambda i: (i, 0)),
        out_shape=jax.ShapeDtypeStruct((_NP, _D), _F32),
    )(a, W, b.reshape(1, _D))


def _lin_logsoftmax(a, Wp, bp):
    def body(a_ref, w_ref, b_ref, o_ref):
        t = jnp.dot(a_ref[...], w_ref[...], preferred_element_type=_F32)
        t = t + b_ref[...]
        m = jnp.max(t, axis=-1, keepdims=True)
        e = jnp.exp(t - m)
        o_ref[...] = (t - m) - jnp.log(jnp.sum(e, axis=-1, keepdims=True))

    return pl.pallas_call(
        body,
        grid=(_NP // _BR,),
        in_specs=[pl.BlockSpec((_BR, _D), lambda i: (i, 0)),
                  pl.BlockSpec((_D, _D), lambda i: (0, 0)),
                  pl.BlockSpec((1, _D), lambda i: (0, 0))],
        out_specs=pl.BlockSpec((_BR, _D), lambda i: (i, 0)),
        out_shape=jax.ShapeDtypeStruct((_NP, _D), _F32),
    )(a, Wp, bp.reshape(1, _D))


def kernel(x, edge_index, pan_w1, W1, b1, pan_w2, W2, b2):
    row = edge_index[0]
    col = edge_index[1]
    padr = jnp.full((_EP - _E,), _N, jnp.int32)
    padc = jnp.zeros((_EP - _E,), jnp.int32)
    rowp = jnp.concatenate([row, padr]).reshape(_NS, _CH, _CW)
    colp = jnp.concatenate([col, padc]).reshape(_NS, _CH, _CW)

    spmm16 = _make_spmm(16, split=False)
    spmm64 = _make_spmm(_DH, split=True)

    ones16 = jnp.ones((_NP, 16), _F32)
    d1w = spmm16(ones16, rowp, colp)
    d1 = d1w[:, 0]
    d2w = spmm16(jnp.broadcast_to(d1[:, None], (_NP, 16)), rowp, colp)
    d2 = d2w[:, 0]

    xp = jnp.pad(x, ((0, _NP - _N), (0, 0)))

    def pan_layer(yp, w):
        deg = w[0] + w[1] * d1 + w[2] * d2
        dinv = jnp.where(deg > 0, lax.rsqrt(deg), 0.0)
        z = dinv[:, None] * yp
        z_lo, z_hi = z[:, :_DH], z[:, _DH:]
        a_lo, a_hi = spmm64(z_lo, z_hi, rowp, colp)
        aa_lo, aa_hi = spmm64(a_lo, a_hi, rowp, colp)
        s_lo = w[0] * z_lo + w[1] * a_lo + w[2] * aa_lo
        s_hi = w[0] * z_hi + w[1] * a_hi + w[2] * aa_hi
        sfull = jnp.concatenate([s_lo, s_hi], axis=1)
        return dinv[:, None] * sfull

    o1 = pan_layer(xp, pan_w1)
    h = _lin_relu(o1, W1, b1)
    o2 = pan_layer(h, pan_w2)
    W2p = jnp.pad(W2, ((0, 0), (0, _D - W2.shape[1])))
    b2p = jnp.concatenate([b2, jnp.full((_D - b2.shape[0],), -1e30, _F32)])
    out = _lin_logsoftmax(o2, W2p, b2p)
    return out[:_N, :W2.shape[1]]


# speedup vs baseline: 5.9266x; 1.2727x over previous
import jax
import jax.numpy as jnp
from jax import lax
from jax.experimental import pallas as pl
from jax.experimental.pallas import tpu as pltpu
from jax.experimental.pallas import tpu_sc as plsc

_N = 10000
_NP = 10240
_E = 320000
_NS = 16
_NC = 2
_CW = 128
_CH = 160
_EP = _NS * _CH * _CW
_RPT = _NP // _NS
_D = 128
_DH = 64
_F32 = jnp.float32


def _mesh():
    return plsc.VectorSubcoreMesh(
        core_axis_name="c", subcore_axis_name="s",
        num_cores=_NC, num_subcores=_NS)


_K = 4
_OB = 128


def _make_spmm(width, split):
    def body(*refs):
        if split:
            ylo, yhi, rowp, colp, olo, ohi = refs[:6]
            rest = refs[6:]
        else:
            y, rowp, colp, out = refs[:4]
            rest = refs[4:]
        idxr, idxc = rest[:2]
        gaths = rest[2:2 + _K]
        obuf, gsem, ssem, acc = rest[2 + _K:]
        c = lax.axis_index("c")
        s = lax.axis_index("s")
        zeros16 = jnp.zeros((16,), _F32)

        pltpu.sync_copy(rowp.at[s], idxr)
        pltpu.sync_copy(colp.at[s], idxc)

        def zrow(i, carry):
            for k in range(width // 16):
                obuf[i, pl.ds(k * 16, 16)] = zeros16
            return carry
        lax.fori_loop(0, _OB, zrow, None)
        for p in range(_RPT // _OB):
            pltpu.sync_copy(obuf, acc.at[pl.ds(s * _RPT + p * _OB, _OB)])
        plsc.subcore_barrier()

        def run(src):
            def group(g, carry):
                base = g * _K
                gd = [pltpu.async_copy(src.at[idxc.at[base + b]], gaths[b],
                                       gsem)
                      for b in range(_K)]
                sd = []
                for b in range(_K):
                    gd[b].wait()
                    sd.append(pltpu.async_copy(gaths[b],
                                               acc.at[idxr.at[base + b]],
                                               ssem, add=True))
                for b in range(_K):
                    sd[b].wait()
                return carry
            lax.fori_loop(0, _CH // _K, group, None)

        if split:
            @pl.when(c == 0)
            def _():
                run(ylo)

            @pl.when(c == 1)
            def _():
                run(yhi)
        else:
            run(y)

        plsc.subcore_barrier()
        if split:
            for p in range(_RPT // _OB):
                pltpu.sync_copy(acc.at[pl.ds(s * _RPT + p * _OB, _OB)], obuf)

                @pl.when(c == 0)
                def _():
                    pltpu.sync_copy(
                        obuf, olo.at[pl.ds(s * _RPT + p * _OB, _OB)])

                @pl.when(c == 1)
                def _():
                    pltpu.sync_copy(
                        obuf, ohi.at[pl.ds(s * _RPT + p * _OB, _OB)])
        else:
            for p in range(_RPT // _OB):
                pltpu.sync_copy(acc.at[pl.ds(s * _RPT + p * _OB, _OB)], obuf)

                @pl.when(c == 0)
                def _():
                    pltpu.sync_copy(
                        obuf, out.at[pl.ds(s * _RPT + p * _OB, _OB)])

    n_out = 2 if split else 1
    out_type = tuple(jax.ShapeDtypeStruct((_NP, width), _F32)
                     for _ in range(n_out))
    if not split:
        out_type = out_type[0]
    return pl.kernel(
        body,
        out_type=out_type,
        mesh=_mesh(),
        compiler_params=pltpu.CompilerParams(use_tc_tiling_on_sc=False),
        scratch_types=[
            pltpu.VMEM((_CH, _CW), jnp.int32),
            pltpu.VMEM((_CH, _CW), jnp.int32),
            *[pltpu.VMEM((_CW, width), _F32) for _ in range(_K)],
            pltpu.VMEM((_OB, width), _F32),
            pltpu.SemaphoreType.DMA,
            pltpu.SemaphoreType.DMA,
            pltpu.VMEM_SHARED((_NP, width), _F32),
        ],
    )


_BR = 1024


def _lin_relu(a, W, b):
    def body(a_ref, w_ref, b_ref, o_ref):
        t = jnp.dot(a_ref[...], w_ref[...], preferred_element_type=_F32)
        o_ref[...] = jnp.maximum(t + b_ref[...], 0.0)

    return pl.pallas_call(
        body,
        grid=(_NP // _BR,),
        in_specs=[pl.BlockSpec((_BR, _D), lambda i: (i, 0)),
                  pl.BlockSpec((_D, _D), lambda i: (0, 0)),
                  pl.BlockSpec((1, _D), lambda i: (0, 0))],
        out_specs=pl.BlockSpec((_BR, _D), lambda i: (i, 0)),
        out_shape=jax.ShapeDtypeStruct((_NP, _D), _F32),
    )(a, W, b.reshape(1, _D))


def _lin_logsoftmax(a, Wp, bp):
    def body(a_ref, w_ref, b_ref, o_ref):
        t = jnp.dot(a_ref[...], w_ref[...], preferred_element_type=_F32)
        t = t + b_ref[...]
        m = jnp.max(t, axis=-1, keepdims=True)
        e = jnp.exp(t - m)
        o_ref[...] = (t - m) - jnp.log(jnp.sum(e, axis=-1, keepdims=True))

    return pl.pallas_call(
        body,
        grid=(_NP // _BR,),
        in_specs=[pl.BlockSpec((_BR, _D), lambda i: (i, 0)),
                  pl.BlockSpec((_D, _D), lambda i: (0, 0)),
                  pl.BlockSpec((1, _D), lambda i: (0, 0))],
        out_specs=pl.BlockSpec((_BR, _D), lambda i: (i, 0)),
        out_shape=jax.ShapeDtypeStruct((_NP, _D), _F32),
    )(a, Wp, bp.reshape(1, _D))


def kernel(x, edge_index, pan_w1, W1, b1, pan_w2, W2, b2):
    row = edge_index[0]
    col = edge_index[1]
    padr = jnp.full((_EP - _E,), _N, jnp.int32)
    padc = jnp.zeros((_EP - _E,), jnp.int32)
    rowp = jnp.concatenate([row, padr]).reshape(_NS, _CH, _CW)
    colp = jnp.concatenate([col, padc]).reshape(_NS, _CH, _CW)

    spmm16 = _make_spmm(16, split=False)
    spmm64 = _make_spmm(_DH, split=True)

    ones16 = jnp.ones((_NP, 16), _F32)
    d1w = spmm16(ones16, rowp, colp)
    d1 = d1w[:, 0]
    d2w = spmm16(jnp.broadcast_to(d1[:, None], (_NP, 16)), rowp, colp)
    d2 = d2w[:, 0]

    xp = jnp.pad(x, ((0, _NP - _N), (0, 0)))

    def pan_layer(yp, w):
        deg = w[0] + w[1] * d1 + w[2] * d2
        dinv = jnp.where(deg > 0, lax.rsqrt(deg), 0.0)
        z = dinv[:, None] * yp
        z_lo, z_hi = z[:, :_DH], z[:, _DH:]
        a_lo, a_hi = spmm64(z_lo, z_hi, rowp, colp)
        aa_lo, aa_hi = spmm64(a_lo, a_hi, rowp, colp)
        s_lo = w[0] * z_lo + w[1] * a_lo + w[2] * aa_lo
        s_hi = w[0] * z_hi + w[1] * a_hi + w[2] * aa_hi
        sfull = jnp.concatenate([s_lo, s_hi], axis=1)
        return dinv[:, None] * sfull

    o1 = pan_layer(xp, pan_w1)
    h = _lin_relu(o1, W1, b1)
    o2 = pan_layer(h, pan_w2)
    W2p = jnp.pad(W2, ((0, 0), (0, _D - W2.shape[1])))
    b2p = jnp.concatenate([b2, jnp.full((_D - b2.shape[0],), -1e30, _F32)])
    out = _lin_logsoftmax(o2, W2p, b2p)
    return out[:_N, :W2.shape[1]]


# speedup vs baseline: 6.1012x; 1.0295x over previous
import jax
import jax.numpy as jnp
from jax import lax
from jax.experimental import pallas as pl
from jax.experimental.pallas import tpu as pltpu
from jax.experimental.pallas import tpu_sc as plsc

_N = 10000
_NP = 10240
_E = 320000
_NS = 16
_NC = 2
_CW = 128
_CH = 160
_EP = _NS * _CH * _CW
_RPT = _NP // _NS
_D = 128
_DH = 64
_F32 = jnp.float32


def _mesh():
    return plsc.VectorSubcoreMesh(
        core_axis_name="c", subcore_axis_name="s",
        num_cores=_NC, num_subcores=_NS)


_K = 4
_OB = 128


def _make_spmm(width, split):
    def body(*refs):
        if split:
            ylo, yhi, rowp, colp, olo, ohi = refs[:6]
            rest = refs[6:]
        else:
            y, rowp, colp, out = refs[:4]
            rest = refs[4:]
        idxr, idxc = rest[:2]
        gaths = rest[2:2 + _K]
        obuf, gsem, ssem, acc = rest[2 + _K:]
        c = lax.axis_index("c")
        s = lax.axis_index("s")
        zeros16 = jnp.zeros((16,), _F32)

        pltpu.sync_copy(rowp.at[s], idxr)
        pltpu.sync_copy(colp.at[s], idxc)

        def zrow(i, carry):
            for k in range(width // 16):
                obuf[i, pl.ds(k * 16, 16)] = zeros16
            return carry
        lax.fori_loop(0, _OB, zrow, None)
        for p in range(_RPT // _OB):
            pltpu.sync_copy(obuf, acc.at[pl.ds(s * _RPT + p * _OB, _OB)])
        plsc.subcore_barrier()

        def run(src):
            ng = _CH // _K

            def group(g, carry):
                base = g * _K
                for b in range(_K):
                    @pl.when(g > 0)
                    def _():
                        pltpu.make_async_copy(
                            gaths[b], acc.at[idxr.at[base - _K + b]],
                            ssem).wait()
                    pltpu.async_copy(src.at[idxc.at[base + b]], gaths[b],
                                     gsem)
                for b in range(_K):
                    pltpu.make_async_copy(src.at[idxc.at[base + b]],
                                          gaths[b], gsem).wait()
                    pltpu.async_copy(gaths[b], acc.at[idxr.at[base + b]],
                                     ssem, add=True)
                return carry
            lax.fori_loop(0, ng, group, None)
            for b in range(_K):
                pltpu.make_async_copy(
                    gaths[b], acc.at[idxr.at[(ng - 1) * _K + b]],
                    ssem).wait()

        if split:
            @pl.when(c == 0)
            def _():
                run(ylo)

            @pl.when(c == 1)
            def _():
                run(yhi)
        else:
            run(y)

        plsc.subcore_barrier()
        if split:
            for p in range(_RPT // _OB):
                pltpu.sync_copy(acc.at[pl.ds(s * _RPT + p * _OB, _OB)], obuf)

                @pl.when(c == 0)
                def _():
                    pltpu.sync_copy(
                        obuf, olo.at[pl.ds(s * _RPT + p * _OB, _OB)])

                @pl.when(c == 1)
                def _():
                    pltpu.sync_copy(
                        obuf, ohi.at[pl.ds(s * _RPT + p * _OB, _OB)])
        else:
            for p in range(_RPT // _OB):
                pltpu.sync_copy(acc.at[pl.ds(s * _RPT + p * _OB, _OB)], obuf)

                @pl.when(c == 0)
                def _():
                    pltpu.sync_copy(
                        obuf, out.at[pl.ds(s * _RPT + p * _OB, _OB)])

    n_out = 2 if split else 1
    out_type = tuple(jax.ShapeDtypeStruct((_NP, width), _F32)
                     for _ in range(n_out))
    if not split:
        out_type = out_type[0]
    return pl.kernel(
        body,
        out_type=out_type,
        mesh=_mesh(),
        compiler_params=pltpu.CompilerParams(use_tc_tiling_on_sc=False),
        scratch_types=[
            pltpu.VMEM((_CH, _CW), jnp.int32),
            pltpu.VMEM((_CH, _CW), jnp.int32),
            *[pltpu.VMEM((_CW, width), _F32) for _ in range(_K)],
            pltpu.VMEM((_OB, width), _F32),
            pltpu.SemaphoreType.DMA,
            pltpu.SemaphoreType.DMA,
            pltpu.VMEM_SHARED((_NP, width), _F32),
        ],
    )


_BR = 1024


def _lin_relu(a, W, b):
    def body(a_ref, w_ref, b_ref, o_ref):
        t = jnp.dot(a_ref[...], w_ref[...], preferred_element_type=_F32)
        o_ref[...] = jnp.maximum(t + b_ref[...], 0.0)

    return pl.pallas_call(
        body,
        grid=(_NP // _BR,),
        in_specs=[pl.BlockSpec((_BR, _D), lambda i: (i, 0)),
                  pl.BlockSpec((_D, _D), lambda i: (0, 0)),
                  pl.BlockSpec((1, _D), lambda i: (0, 0))],
        out_specs=pl.BlockSpec((_BR, _D), lambda i: (i, 0)),
        out_shape=jax.ShapeDtypeStruct((_NP, _D), _F32),
    )(a, W, b.reshape(1, _D))


def _lin_logsoftmax(a, Wp, bp):
    def body(a_ref, w_ref, b_ref, o_ref):
        t = jnp.dot(a_ref[...], w_ref[...], preferred_element_type=_F32)
        t = t + b_ref[...]
        m = jnp.max(t, axis=-1, keepdims=True)
        e = jnp.exp(t - m)
        o_ref[...] = (t - m) - jnp.log(jnp.sum(e, axis=-1, keepdims=True))

    return pl.pallas_call(
        body,
        grid=(_NP // _BR,),
        in_specs=[pl.BlockSpec((_BR, _D), lambda i: (i, 0)),
                  pl.BlockSpec((_D, _D), lambda i: (0, 0)),
                  pl.BlockSpec((1, _D), lambda i: (0, 0))],
        out_specs=pl.BlockSpec((_BR, _D), lambda i: (i, 0)),
        out_shape=jax.ShapeDtypeStruct((_NP, _D), _F32),
    )(a, Wp, bp.reshape(1, _D))


def kernel(x, edge_index, pan_w1, W1, b1, pan_w2, W2, b2):
    row = edge_index[0]
    col = edge_index[1]
    padr = jnp.full((_EP - _E,), _N, jnp.int32)
    padc = jnp.zeros((_EP - _E,), jnp.int32)
    rowp = jnp.concatenate([row, padr]).reshape(_NS, _CH, _CW)
    colp = jnp.concatenate([col, padc]).reshape(_NS, _CH, _CW)

    spmm16 = _make_spmm(16, split=False)
    spmm64 = _make_spmm(_DH, split=True)

    ones16 = jnp.ones((_NP, 16), _F32)
    d1w = spmm16(ones16, rowp, colp)
    d1 = d1w[:, 0]
    d2w = spmm16(jnp.broadcast_to(d1[:, None], (_NP, 16)), rowp, colp)
    d2 = d2w[:, 0]

    xp = jnp.pad(x, ((0, _NP - _N), (0, 0)))

    def pan_layer(yp, w):
        deg = w[0] + w[1] * d1 + w[2] * d2
        dinv = jnp.where(deg > 0, lax.rsqrt(deg), 0.0)
        z = dinv[:, None] * yp
        z_lo, z_hi = z[:, :_DH], z[:, _DH:]
        a_lo, a_hi = spmm64(z_lo, z_hi, rowp, colp)
        aa_lo, aa_hi = spmm64(a_lo, a_hi, rowp, colp)
        s_lo = w[0] * z_lo + w[1] * a_lo + w[2] * aa_lo
        s_hi = w[0] * z_hi + w[1] * a_hi + w[2] * aa_hi
        sfull = jnp.concatenate([s_lo, s_hi], axis=1)
        return dinv[:, None] * sfull

    o1 = pan_layer(xp, pan_w1)
    h = _lin_relu(o1, W1, b1)
    o2 = pan_layer(h, pan_w2)
    W2p = jnp.pad(W2, ((0, 0), (0, _D - W2.shape[1])))
    b2p = jnp.concatenate([b2, jnp.full((_D - b2.shape[0],), -1e30, _F32)])
    out = _lin_logsoftmax(o2, W2p, b2p)
    return out[:_N, :W2.shape[1]]


# speedup vs baseline: 6.9578x; 1.1404x over previous
import jax
import jax.numpy as jnp
from jax import lax
from jax.experimental import pallas as pl
from jax.experimental.pallas import tpu as pltpu
from jax.experimental.pallas import tpu_sc as plsc

_N = 10000
_NP = 10240
_E = 320000
_NS = 16
_NC = 2
_CW = 128
_CH = 160
_EP = _NS * _CH * _CW
_RPT = _NP // _NS
_D = 128
_DH = 64
_F32 = jnp.float32


def _mesh():
    return plsc.VectorSubcoreMesh(
        core_axis_name="c", subcore_axis_name="s",
        num_cores=_NC, num_subcores=_NS)


_K = 4
_OB = 128


def _make_spmm(width, split):
    def body(*refs):
        if split:
            ylo, yhi, rowp, colp, olo, ohi = refs[:6]
            rest = refs[6:]
        else:
            y, rowp, colp, out = refs[:4]
            rest = refs[4:]
        idxr, idxc = rest[:2]
        gaths = rest[2:2 + _K]
        obuf, gsem, ssem, acc = rest[2 + _K:]
        c = lax.axis_index("c")
        s = lax.axis_index("s")
        zeros16 = jnp.zeros((16,), _F32)

        pltpu.sync_copy(rowp.at[s], idxr)
        pltpu.sync_copy(colp.at[s], idxc)

        def zrow(i, carry):
            for k in range(width // 16):
                obuf[i, pl.ds(k * 16, 16)] = zeros16
            return carry
        lax.fori_loop(0, _OB, zrow, None)
        for p in range(_RPT // _OB):
            pltpu.sync_copy(obuf, acc.at[pl.ds(s * _RPT + p * _OB, _OB)])
        plsc.subcore_barrier()

        def run(src):
            ng = _CH // _K

            def group(g, carry):
                base = g * _K
                for b in range(_K):
                    @pl.when(g > 0)
                    def _():
                        pltpu.make_async_copy(
                            gaths[b], acc.at[idxr.at[base - _K + b]],
                            ssem).wait()
                    pltpu.async_copy(src.at[idxc.at[base + b]], gaths[b],
                                     gsem)
                for b in range(_K):
                    pltpu.make_async_copy(src.at[idxc.at[base + b]],
                                          gaths[b], gsem).wait()
                    pltpu.async_copy(gaths[b], acc.at[idxr.at[base + b]],
                                     ssem, add=True)
                return carry
            lax.fori_loop(0, ng, group, None)
            for b in range(_K):
                pltpu.make_async_copy(
                    gaths[b], acc.at[idxr.at[(ng - 1) * _K + b]],
                    ssem).wait()

        if split:
            @pl.when(c == 0)
            def _():
                run(ylo)

            @pl.when(c == 1)
            def _():
                run(yhi)
        else:
            run(y)

        plsc.subcore_barrier()
        if split:
            for p in range(_RPT // _OB):
                pltpu.sync_copy(acc.at[pl.ds(s * _RPT + p * _OB, _OB)], obuf)

                @pl.when(c == 0)
                def _():
                    pltpu.sync_copy(
                        obuf, olo.at[pl.ds(s * _RPT + p * _OB, _OB)])

                @pl.when(c == 1)
                def _():
                    pltpu.sync_copy(
                        obuf, ohi.at[pl.ds(s * _RPT + p * _OB, _OB)])
        else:
            for p in range(_RPT // _OB):
                pltpu.sync_copy(acc.at[pl.ds(s * _RPT + p * _OB, _OB)], obuf)

                @pl.when(c == 0)
                def _():
                    pltpu.sync_copy(
                        obuf, out.at[pl.ds(s * _RPT + p * _OB, _OB)])

    n_out = 2 if split else 1
    out_type = tuple(jax.ShapeDtypeStruct((_NP, width), _F32)
                     for _ in range(n_out))
    if not split:
        out_type = out_type[0]
    return _spmm_kernel(body, out_type, width)


def _spmm_kernel(body, out_type, width):
    return pl.kernel(
        body,
        out_type=out_type,
        mesh=_mesh(),
        compiler_params=pltpu.CompilerParams(use_tc_tiling_on_sc=False),
        scratch_types=[
            pltpu.VMEM((_CH, _CW), jnp.int32),
            pltpu.VMEM((_CH, _CW), jnp.int32),
            *[pltpu.VMEM((_CW, width), _F32) for _ in range(_K)],
            pltpu.VMEM((_OB, width), _F32),
            pltpu.SemaphoreType.DMA,
            pltpu.SemaphoreType.DMA,
            pltpu.VMEM_SHARED((_NP, width), _F32),
        ],
    )


def _deg_call(rowp, colp):
    def body(*refs):
        (rowp_h, colp_h, d1o, d2o, idxr, idxc, ones_b) = refs[:7]
        gaths = refs[7:7 + _K]
        obuf, gsem, ssem, acc1, acc2 = refs[7 + _K:]
        c = lax.axis_index("c")
        s = lax.axis_index("s")
        zeros16 = jnp.zeros((16,), _F32)
        ones16 = jnp.ones((16,), _F32)

        pltpu.sync_copy(rowp_h.at[s], idxr)
        pltpu.sync_copy(colp_h.at[s], idxc)

        def fill(i, carry):
            obuf[i, pl.ds(0, 16)] = zeros16
            ones_b[i, pl.ds(0, 16)] = ones16
            return carry
        lax.fori_loop(0, _OB, fill, None)
        for p in range(_RPT // _OB):
            pltpu.sync_copy(obuf, acc1.at[pl.ds(s * _RPT + p * _OB, _OB)])
            pltpu.sync_copy(obuf, acc2.at[pl.ds(s * _RPT + p * _OB, _OB)])
        plsc.subcore_barrier()

        ng = _CH // _K

        def grp1(g, carry):
            base = g * _K
            for b in range(_K):
                @pl.when(g > 0)
                def _():
                    pltpu.make_async_copy(
                        ones_b, acc1.at[idxr.at[base - _K + b]], ssem).wait()
                pltpu.async_copy(ones_b, acc1.at[idxr.at[base + b]], ssem,
                                 add=True)
            return carry
        lax.fori_loop(0, ng, grp1, None)
        for b in range(_K):
            pltpu.make_async_copy(
                ones_b, acc1.at[idxr.at[(ng - 1) * _K + b]], ssem).wait()
        plsc.subcore_barrier()

        def grp2(g, carry):
            base = g * _K
            for b in range(_K):
                @pl.when(g > 0)
                def _():
                    pltpu.make_async_copy(
                        gaths[b], acc2.at[idxr.at[base - _K + b]],
                        ssem).wait()
                pltpu.async_copy(acc1.at[idxc.at[base + b]], gaths[b], gsem)
            for b in range(_K):
                pltpu.make_async_copy(acc1.at[idxc.at[base + b]], gaths[b],
                                      gsem).wait()
                pltpu.async_copy(gaths[b], acc2.at[idxr.at[base + b]], ssem,
                                 add=True)
            return carry
        lax.fori_loop(0, ng, grp2, None)
        for b in range(_K):
            pltpu.make_async_copy(
                gaths[b], acc2.at[idxr.at[(ng - 1) * _K + b]], ssem).wait()
        plsc.subcore_barrier()

        @pl.when(c == 0)
        def _():
            for p in range(_RPT // _OB):
                sl = pl.ds(s * _RPT + p * _OB, _OB)
                pltpu.sync_copy(acc1.at[sl], obuf)
                pltpu.sync_copy(obuf, d1o.at[sl])
                pltpu.sync_copy(acc2.at[sl], obuf)
                pltpu.sync_copy(obuf, d2o.at[sl])

    f = pl.kernel(
        body,
        out_type=(jax.ShapeDtypeStruct((_NP, 16), _F32),
                  jax.ShapeDtypeStruct((_NP, 16), _F32)),
        mesh=_mesh(),
        compiler_params=pltpu.CompilerParams(use_tc_tiling_on_sc=False),
        scratch_types=[
            pltpu.VMEM((_CH, _CW), jnp.int32),
            pltpu.VMEM((_CH, _CW), jnp.int32),
            pltpu.VMEM((_CW, 16), _F32),
            *[pltpu.VMEM((_CW, 16), _F32) for _ in range(_K)],
            pltpu.VMEM((_OB, 16), _F32),
            pltpu.SemaphoreType.DMA,
            pltpu.SemaphoreType.DMA,
            pltpu.VMEM_SHARED((_NP, 16), _F32),
            pltpu.VMEM_SHARED((_NP, 16), _F32),
        ],
    )
    return f(rowp, colp)


_BR = 1024


def _lin_relu(a, W, b):
    def body(a_ref, w_ref, b_ref, o_ref):
        t = jnp.dot(a_ref[...], w_ref[...], preferred_element_type=_F32)
        o_ref[...] = jnp.maximum(t + b_ref[...], 0.0)

    return pl.pallas_call(
        body,
        grid=(_NP // _BR,),
        in_specs=[pl.BlockSpec((_BR, _D), lambda i: (i, 0)),
                  pl.BlockSpec((_D, _D), lambda i: (0, 0)),
                  pl.BlockSpec((1, _D), lambda i: (0, 0))],
        out_specs=pl.BlockSpec((_BR, _D), lambda i: (i, 0)),
        out_shape=jax.ShapeDtypeStruct((_NP, _D), _F32),
    )(a, W, b.reshape(1, _D))


def _lin_logsoftmax(a, Wp, bp):
    def body(a_ref, w_ref, b_ref, o_ref):
        t = jnp.dot(a_ref[...], w_ref[...], preferred_element_type=_F32)
        t = t + b_ref[...]
        m = jnp.max(t, axis=-1, keepdims=True)
        e = jnp.exp(t - m)
        o_ref[...] = (t - m) - jnp.log(jnp.sum(e, axis=-1, keepdims=True))

    return pl.pallas_call(
        body,
        grid=(_NP // _BR,),
        in_specs=[pl.BlockSpec((_BR, _D), lambda i: (i, 0)),
                  pl.BlockSpec((_D, _D), lambda i: (0, 0)),
                  pl.BlockSpec((1, _D), lambda i: (0, 0))],
        out_specs=pl.BlockSpec((_BR, _D), lambda i: (i, 0)),
        out_shape=jax.ShapeDtypeStruct((_NP, _D), _F32),
    )(a, Wp, bp.reshape(1, _D))


def kernel(x, edge_index, pan_w1, W1, b1, pan_w2, W2, b2):
    row = edge_index[0]
    col = edge_index[1]
    padr = jnp.full((_EP - _E,), _N, jnp.int32)
    padc = jnp.zeros((_EP - _E,), jnp.int32)
    rowp = jnp.concatenate([row, padr]).reshape(_NS, _CH, _CW)
    colp = jnp.concatenate([col, padc]).reshape(_NS, _CH, _CW)

    spmm64 = _make_spmm(_DH, split=True)

    d1w, d2w = _deg_call(rowp, colp)
    d1 = d1w[:, 0]
    d2 = d2w[:, 0]

    xp = jnp.pad(x, ((0, _NP - _N), (0, 0)))

    def pan_layer(yp, w):
        deg = w[0] + w[1] * d1 + w[2] * d2
        dinv = jnp.where(deg > 0, lax.rsqrt(deg), 0.0)
        z = dinv[:, None] * yp
        z_lo, z_hi = z[:, :_DH], z[:, _DH:]
        a_lo, a_hi = spmm64(z_lo, z_hi, rowp, colp)
        aa_lo, aa_hi = spmm64(a_lo, a_hi, rowp, colp)
        s_lo = w[0] * z_lo + w[1] * a_lo + w[2] * aa_lo
        s_hi = w[0] * z_hi + w[1] * a_hi + w[2] * aa_hi
        sfull = jnp.concatenate([s_lo, s_hi], axis=1)
        return dinv[:, None] * sfull

    o1 = pan_layer(xp, pan_w1)
    h = _lin_relu(o1, W1, b1)
    o2 = pan_layer(h, pan_w2)
    W2p = jnp.pad(W2, ((0, 0), (0, _D - W2.shape[1])))
    b2p = jnp.concatenate([b2, jnp.full((_D - b2.shape[0],), -1e30, _F32)])
    out = _lin_logsoftmax(o2, W2p, b2p)
    return out[:_N, :W2.shape[1]]


# speedup vs baseline: 9.5195x; 1.3682x over previous
import jax
import jax.numpy as jnp
from jax import lax
from jax.experimental import pallas as pl
from jax.experimental.pallas import tpu as pltpu
from jax.experimental.pallas import tpu_sc as plsc

_N = 10000
_NP = 10240
_E = 320000
_NS = 16
_NC = 2
_CW = 128
_CH = 160
_EP = _NS * _CH * _CW
_RPT = _NP // _NS
_D = 128
_DH = 64
_F32 = jnp.float32


def _mesh():
    return plsc.VectorSubcoreMesh(
        core_axis_name="c", subcore_axis_name="s",
        num_cores=_NC, num_subcores=_NS)


_K = 4
_OB = 128


def _layer_call(zlo, zhi, rowp, colp):
    npairs = _CH // _K // 2

    def body(*refs):
        zlo_h, zhi_h, rowp_h, colp_h, alo, ahi, aalo, aahi = refs[:8]
        irA, icA, irB, icB = refs[8:12]
        gaths = refs[12:12 + _K]
        obuf, isem, gsem, ssem, acc1, acc2 = refs[12 + _K:]
        c = lax.axis_index("c")
        s = lax.axis_index("s")
        zeros16 = jnp.zeros((16,), _F32)

        def zrow(i, carry):
            for k in range(_DH // 16):
                obuf[i, pl.ds(k * 16, 16)] = zeros16
            return carry
        lax.fori_loop(0, _OB, zrow, None)
        for p in range(_RPT // _OB):
            pltpu.sync_copy(obuf, acc1.at[pl.ds(s * _RPT + p * _OB, _OB)])
            pltpu.sync_copy(obuf, acc2.at[pl.ds(s * _RPT + p * _OB, _OB)])
        plsc.subcore_barrier()

        def idx_slice(hbm, g):
            return hbm.at[s, pl.ds(g * _K, _K)]

        def stream_pass(src, dst):
            pltpu.sync_copy(idx_slice(rowp_h, 0), irA)
            pltpu.sync_copy(idx_slice(colp_h, 0), icA)
            pltpu.async_copy(idx_slice(rowp_h, 1), irB, isem)
            pltpu.async_copy(idx_slice(colp_h, 1), icB, isem)

            def pair(p, carry):
                g0 = 2 * p
                g1 = g0 + 1

                @pl.when(p > 0)
                def _():
                    pltpu.make_async_copy(idx_slice(rowp_h, g0), irA,
                                          isem).wait()
                    pltpu.make_async_copy(idx_slice(colp_h, g0), icA,
                                          isem).wait()
                for b in range(_K):
                    @pl.when(p > 0)
                    def _():
                        pltpu.make_async_copy(gaths[b], dst.at[irA.at[b]],
                                              ssem).wait()
                    pltpu.async_copy(src.at[icA.at[b]], gaths[b], gsem)

                @pl.when(p > 0)
                def _():
                    pltpu.async_copy(idx_slice(rowp_h, g1), irB, isem)
                    pltpu.async_copy(idx_slice(colp_h, g1), icB, isem)
                for b in range(_K):
                    pltpu.make_async_copy(src.at[icA.at[b]], gaths[b],
                                          gsem).wait()
                    pltpu.async_copy(gaths[b], dst.at[irA.at[b]], ssem,
                                     add=True)

                pltpu.make_async_copy(idx_slice(rowp_h, g1), irB, isem).wait()
                pltpu.make_async_copy(idx_slice(colp_h, g1), icB, isem).wait()
                for b in range(_K):
                    pltpu.make_async_copy(gaths[b], dst.at[irB.at[b]],
                                          ssem).wait()
                    pltpu.async_copy(src.at[icB.at[b]], gaths[b], gsem)

                @pl.when(p < npairs - 1)
                def _():
                    pltpu.async_copy(idx_slice(rowp_h, g0 + 2), irA, isem)
                    pltpu.async_copy(idx_slice(colp_h, g0 + 2), icA, isem)
                for b in range(_K):
                    pltpu.make_async_copy(src.at[icB.at[b]], gaths[b],
                                          gsem).wait()
                    pltpu.async_copy(gaths[b], dst.at[irB.at[b]], ssem,
                                     add=True)
                return carry
            lax.fori_loop(0, npairs, pair, None)
            for b in range(_K):
                pltpu.make_async_copy(gaths[b], dst.at[irB.at[b]],
                                      ssem).wait()

        @pl.when(c == 0)
        def _():
            stream_pass(zlo_h, acc1)

        @pl.when(c == 1)
        def _():
            stream_pass(zhi_h, acc1)

        plsc.subcore_barrier()
        stream_pass(acc1, acc2)
        plsc.subcore_barrier()

        def copy_out(acc, olo, ohi):
            for p in range(_RPT // _OB):
                sl = pl.ds(s * _RPT + p * _OB, _OB)
                pltpu.sync_copy(acc.at[sl], obuf)

                @pl.when(c == 0)
                def _():
                    pltpu.sync_copy(obuf, olo.at[sl])

                @pl.when(c == 1)
                def _():
                    pltpu.sync_copy(obuf, ohi.at[sl])

        copy_out(acc1, alo, ahi)
        copy_out(acc2, aalo, aahi)

    f = pl.kernel(
        body,
        out_type=tuple(jax.ShapeDtypeStruct((_NP, _DH), _F32)
                       for _ in range(4)),
        mesh=_mesh(),
        compiler_params=pltpu.CompilerParams(use_tc_tiling_on_sc=False),
        scratch_types=[
            *[pltpu.VMEM((_K, _CW), jnp.int32) for _ in range(4)],
            *[pltpu.VMEM((_CW, _DH), _F32) for _ in range(_K)],
            pltpu.VMEM((_OB, _DH), _F32),
            pltpu.SemaphoreType.DMA,
            pltpu.SemaphoreType.DMA,
            pltpu.SemaphoreType.DMA,
            pltpu.VMEM_SHARED((_NP, _DH), _F32),
            pltpu.VMEM_SHARED((_NP, _DH), _F32),
        ],
    )
    return f(zlo, zhi, rowp, colp)


def _deg_call(rowp, colp):
    def body(*refs):
        (rowp_h, colp_h, d1o, d2o, idxr, idxc, ones_b) = refs[:7]
        gaths = refs[7:7 + _K]
        obuf, gsem, ssem, acc1, acc2 = refs[7 + _K:]
        c = lax.axis_index("c")
        s = lax.axis_index("s")
        zeros16 = jnp.zeros((16,), _F32)
        ones16 = jnp.ones((16,), _F32)

        pltpu.sync_copy(rowp_h.at[s], idxr)
        pltpu.sync_copy(colp_h.at[s], idxc)

        def fill(i, carry):
            obuf[i, pl.ds(0, 16)] = zeros16
            ones_b[i, pl.ds(0, 16)] = ones16
            return carry
        lax.fori_loop(0, _OB, fill, None)
        for p in range(_RPT // _OB):
            pltpu.sync_copy(obuf, acc1.at[pl.ds(s * _RPT + p * _OB, _OB)])
            pltpu.sync_copy(obuf, acc2.at[pl.ds(s * _RPT + p * _OB, _OB)])
        plsc.subcore_barrier()

        ng = _CH // _K

        def grp1(g, carry):
            base = g * _K
            for b in range(_K):
                @pl.when(g > 0)
                def _():
                    pltpu.make_async_copy(
                        ones_b, acc1.at[idxr.at[base - _K + b]], ssem).wait()
                pltpu.async_copy(ones_b, acc1.at[idxr.at[base + b]], ssem,
                                 add=True)
            return carry
        lax.fori_loop(0, ng, grp1, None)
        for b in range(_K):
            pltpu.make_async_copy(
                ones_b, acc1.at[idxr.at[(ng - 1) * _K + b]], ssem).wait()
        plsc.subcore_barrier()

        def grp2(g, carry):
            base = g * _K
            for b in range(_K):
                @pl.when(g > 0)
                def _():
                    pltpu.make_async_copy(
                        gaths[b], acc2.at[idxr.at[base - _K + b]],
                        ssem).wait()
                pltpu.async_copy(acc1.at[idxc.at[base + b]], gaths[b], gsem)
            for b in range(_K):
                pltpu.make_async_copy(acc1.at[idxc.at[base + b]], gaths[b],
                                      gsem).wait()
                pltpu.async_copy(gaths[b], acc2.at[idxr.at[base + b]], ssem,
                                 add=True)
            return carry
        lax.fori_loop(0, ng, grp2, None)
        for b in range(_K):
            pltpu.make_async_copy(
                gaths[b], acc2.at[idxr.at[(ng - 1) * _K + b]], ssem).wait()
        plsc.subcore_barrier()

        @pl.when(c == 0)
        def _():
            for p in range(_RPT // _OB):
                sl = pl.ds(s * _RPT + p * _OB, _OB)
                pltpu.sync_copy(acc1.at[sl], obuf)
                pltpu.sync_copy(obuf, d1o.at[sl])
                pltpu.sync_copy(acc2.at[sl], obuf)
                pltpu.sync_copy(obuf, d2o.at[sl])

    f = pl.kernel(
        body,
        out_type=(jax.ShapeDtypeStruct((_NP, 16), _F32),
                  jax.ShapeDtypeStruct((_NP, 16), _F32)),
        mesh=_mesh(),
        compiler_params=pltpu.CompilerParams(use_tc_tiling_on_sc=False),
        scratch_types=[
            pltpu.VMEM((_CH, _CW), jnp.int32),
            pltpu.VMEM((_CH, _CW), jnp.int32),
            pltpu.VMEM((_CW, 16), _F32),
            *[pltpu.VMEM((_CW, 16), _F32) for _ in range(_K)],
            pltpu.VMEM((_OB, 16), _F32),
            pltpu.SemaphoreType.DMA,
            pltpu.SemaphoreType.DMA,
            pltpu.VMEM_SHARED((_NP, 16), _F32),
            pltpu.VMEM_SHARED((_NP, 16), _F32),
        ],
    )
    return f(rowp, colp)


_BR = 1024


def _lin_relu(a, W, b):
    def body(a_ref, w_ref, b_ref, o_ref):
        t = jnp.dot(a_ref[...], w_ref[...], preferred_element_type=_F32)
        o_ref[...] = jnp.maximum(t + b_ref[...], 0.0)

    return pl.pallas_call(
        body,
        grid=(_NP // _BR,),
        in_specs=[pl.BlockSpec((_BR, _D), lambda i: (i, 0)),
                  pl.BlockSpec((_D, _D), lambda i: (0, 0)),
                  pl.BlockSpec((1, _D), lambda i: (0, 0))],
        out_specs=pl.BlockSpec((_BR, _D), lambda i: (i, 0)),
        out_shape=jax.ShapeDtypeStruct((_NP, _D), _F32),
    )(a, W, b.reshape(1, _D))


def _lin_logsoftmax(a, Wp, bp):
    def body(a_ref, w_ref, b_ref, o_ref):
        t = jnp.dot(a_ref[...], w_ref[...], preferred_element_type=_F32)
        t = t + b_ref[...]
        m = jnp.max(t, axis=-1, keepdims=True)
        e = jnp.exp(t - m)
        o_ref[...] = (t - m) - jnp.log(jnp.sum(e, axis=-1, keepdims=True))

    return pl.pallas_call(
        body,
        grid=(_NP // _BR,),
        in_specs=[pl.BlockSpec((_BR, _D), lambda i: (i, 0)),
                  pl.BlockSpec((_D, _D), lambda i: (0, 0)),
                  pl.BlockSpec((1, _D), lambda i: (0, 0))],
        out_specs=pl.BlockSpec((_BR, _D), lambda i: (i, 0)),
        out_shape=jax.ShapeDtypeStruct((_NP, _D), _F32),
    )(a, Wp, bp.reshape(1, _D))


def kernel(x, edge_index, pan_w1, W1, b1, pan_w2, W2, b2):
    row = edge_index[0]
    col = edge_index[1]
    padr = jnp.full((_EP - _E,), _N, jnp.int32)
    padc = jnp.zeros((_EP - _E,), jnp.int32)
    rowp = jnp.concatenate([row, padr]).reshape(_NS, _CH, _CW)
    colp = jnp.concatenate([col, padc]).reshape(_NS, _CH, _CW)

    d1w, d2w = _deg_call(rowp, colp)
    d1 = d1w[:, 0]
    d2 = d2w[:, 0]

    xp = jnp.pad(x, ((0, _NP - _N), (0, 0)))

    def pan_layer(yp, w):
        deg = w[0] + w[1] * d1 + w[2] * d2
        dinv = jnp.where(deg > 0, lax.rsqrt(deg), 0.0)
        z = dinv[:, None] * yp
        z_lo, z_hi = z[:, :_DH], z[:, _DH:]
        a_lo, a_hi, aa_lo, aa_hi = _layer_call(z_lo, z_hi, rowp, colp)
        s_lo = w[0] * z_lo + w[1] * a_lo + w[2] * aa_lo
        s_hi = w[0] * z_hi + w[1] * a_hi + w[2] * aa_hi
        sfull = jnp.concatenate([s_lo, s_hi], axis=1)
        return dinv[:, None] * sfull

    o1 = pan_layer(xp, pan_w1)
    h = _lin_relu(o1, W1, b1)
    o2 = pan_layer(h, pan_w2)
    W2p = jnp.pad(W2, ((0, 0), (0, _D - W2.shape[1])))
    b2p = jnp.concatenate([b2, jnp.full((_D - b2.shape[0],), -1e30, _F32)])
    out = _lin_logsoftmax(o2, W2p, b2p)
    return out[:_N, :W2.shape[1]]
